# baseline (speedup 1.0000x reference)
import functools

import jax
import jax.numpy as jnp
from jax import lax
from jax.experimental import pallas as pl
from jax.experimental.pallas import tpu as pltpu
from jax.experimental.pallas import tpu_sc as plsc

DIM = 64
BATCH = 16384
NC = 2
NS = 16
NW = NC * NS
BPW = BATCH // NW
IDX_MINOR = 128
N_CHUNKS = BPW // IDX_MINOR

_mesh = plsc.VectorSubcoreMesh(core_axis_name="c", subcore_axis_name="s")


@functools.partial(
    pl.kernel,
    mesh=_mesh,
    compiler_params=pltpu.CompilerParams(
        needs_layout_passes=False, use_tc_tiling_on_sc=False),
    out_type=jax.ShapeDtypeStruct((BATCH,), jnp.float32),
    scratch_types=[
        pltpu.VMEM((N_CHUNKS, IDX_MINOR), jnp.int32),
        pltpu.VMEM((N_CHUNKS, IDX_MINOR), jnp.int32),
        pltpu.VMEM((BPW, DIM), jnp.float32),
        pltpu.VMEM((BPW, DIM), jnp.float32),
        pltpu.VMEM((BPW,), jnp.float32),
        pltpu.SemaphoreType.DMA,
        pltpu.SemaphoreType.DMA,
    ],
)
def _sc_dot_gather(users_hbm, items_hbm, utab_hbm, itab_hbm, out_hbm,
                   uidx_v, iidx_v, urows_v, irows_v, sc_v, usem, isem):
    wid = lax.axis_index("s") * NC + lax.axis_index("c")
    base = wid * BPW

    for j in range(N_CHUNKS):
        pltpu.sync_copy(users_hbm.at[pl.ds(base + j * IDX_MINOR, IDX_MINOR)],
                        uidx_v.at[j])
        pltpu.sync_copy(items_hbm.at[pl.ds(base + j * IDX_MINOR, IDX_MINOR)],
                        iidx_v.at[j])

    ucopies = []
    icopies = []
    for j in range(N_CHUNKS):
        ucopies.append(pltpu.async_copy(
            utab_hbm.at[uidx_v.at[j]],
            urows_v.at[pl.ds(j * IDX_MINOR, IDX_MINOR)], usem))
        icopies.append(pltpu.async_copy(
            itab_hbm.at[iidx_v.at[j]],
            irows_v.at[pl.ds(j * IDX_MINOR, IDX_MINOR)], isem))
    for c in ucopies:
        c.wait()
    for c in icopies:
        c.wait()

    lanes = lax.iota(jnp.int32, 16)

    def group_body(g, carry):
        row_idx = g * 16 + lanes
        acc = jnp.zeros((16,), jnp.float32)
        for d in range(DIM):
            col = jnp.full((16,), d, jnp.int32)
            uv = plsc.load_gather(urows_v, [row_idx, col])
            iv = plsc.load_gather(irows_v, [row_idx, col])
            acc = acc + uv * iv
        sc_v[pl.ds(g * 16, 16)] = acc
        return carry

    lax.fori_loop(0, BPW // 16, group_body, 0)

    pltpu.sync_copy(sc_v, out_hbm.at[pl.ds(base, BPW)])


def kernel(users, items, user_table, item_table):
    users = users.astype(jnp.int32)
    items = items.astype(jnp.int32)
    return _sc_dot_gather(users, items, user_table, item_table)
